# baseline (speedup 1.0000x reference)
import functools

import jax
import jax.numpy as jnp
from jax import lax
from jax.experimental import pallas as pl
from jax.experimental.pallas import tpu as pltpu
from jax.experimental.pallas import tpu_sc as plsc

F = 26
V = 100000
E = 16
B = 16384
NC = 2
NS = 16
NW = NC * NS
BPW = B // NW
CB = 64
NCH = BPW // CB
ROWS = F * CB
IDX_MINOR = 128
IDX_ROWS = ROWS // IDX_MINOR


def _worker_id():
  return lax.axis_index("s") * NC + lax.axis_index("c")


def _colsum(tbuf):
  riota = lax.iota(jnp.int32, 16)
  parts = []
  for e0 in range(0, 16, 4):
    p = plsc.load_gather(tbuf, [riota, jnp.full((16,), e0, jnp.int32)])
    for e in range(e0 + 1, e0 + 4):
      p = p + plsc.load_gather(tbuf, [riota, jnp.full((16,), e, jnp.int32)])
    parts.append(p)
  return (parts[0] + parts[1]) + (parts[2] + parts[3])


def _fire(tab_hbm, idx_ref, d, sem):
  pltpu.async_copy(tab_hbm.at[idx_ref], d, sem)


def _wait(tab_hbm, idx_ref, d, sem):
  pltpu.make_async_copy(tab_hbm.at[idx_ref], d, sem).wait()


def _worker_body(xp_hbm, tab_hbm, w_hbm, bv_hbm, out_hbm,
                 xv, idx0, idx1, d0, d1, wvv, bvv, tbuf, outb, sem0, sem1):
  wid = _worker_id()
  base = wid * BPW

  pltpu.sync_copy(xp_hbm.at[wid], xv)
  pltpu.sync_copy(w_hbm, wvv)
  pltpu.sync_copy(bv_hbm, bvv)

  bias = bvv[...]

  def build(idxd, c):
    @pl.loop(0, F)
    def _(f):
      for v in range(CB // 16):
        vec = xv[f, pl.ds(c * CB + v * 16, 16)] + f * V
        idxd[pl.ds(f * CB + v * 16, 16)] = vec

  def fire(idxd, d, sem):
    _fire(tab_hbm, idxd, d, sem)

  def wait(idxd, d, sem):
    _wait(tab_hbm, idxd, d, sem)

  def compute(d, c):
    for jblk in range(CB // 16):
      jb = jblk * 16
      zeros = jnp.zeros((16,), jnp.float32)
      init = (zeros,) * 16

      @pl.loop(0, F, init_carry=init, unroll=2)
      def accs(f, carry):
        wv = wvv[f]
        return tuple(carry[i] + d[f * CB + jb + i] * wv for i in range(16))

      for i in range(16):
        tbuf[i] = accs[i]
      ov = _colsum(tbuf) + bias
      outb[pl.ds(c * CB + jb, 16)] = ov

  build(idx0, 0)
  fire(idx0, d0, sem0)

  @pl.loop(0, NCH, step=2)
  def _main(c):
    wait(idx0, d0, sem0)
    build(idx1, c + 1)
    fire(idx1, d1, sem1)
    compute(d0, c)
    wait(idx1, d1, sem1)

    @pl.when(c + 2 < NCH)
    def _():
      build(idx0, c + 2)
      fire(idx0, d0, sem0)

    compute(d1, c + 1)

  pltpu.sync_copy(outb, out_hbm.at[pl.ds(base, BPW)])


@jax.jit
def _run(xp, tab, w, bv):
  mesh = plsc.VectorSubcoreMesh(
      core_axis_name="c", subcore_axis_name="s",
      num_cores=NC, num_subcores=NS)
  kern = pl.kernel(
      _worker_body,
      out_type=jax.ShapeDtypeStruct((B,), jnp.float32),
      mesh=mesh,
      scratch_types=[
          pltpu.VMEM((F, BPW), jnp.int32),
          pltpu.VMEM((ROWS,), jnp.int32),
          pltpu.VMEM((ROWS,), jnp.int32),
          pltpu.VMEM((ROWS, E), jnp.float32),
          pltpu.VMEM((ROWS, E), jnp.float32),
          pltpu.VMEM((F, E), jnp.float32),
          pltpu.VMEM((E,), jnp.float32),
          pltpu.VMEM((16, 16), jnp.float32),
          pltpu.VMEM((BPW,), jnp.float32),
          pltpu.SemaphoreType.DMA,
          pltpu.SemaphoreType.DMA,
      ],
      compiler_params=pltpu.CompilerParams(
          needs_layout_passes=False, use_tc_tiling_on_sc=False),
  )
  return kern(xp, tab, w, bv)


def kernel(x, tables, W, b):
  x = x.astype(jnp.int32)
  xp = x.reshape(NW, BPW, F).transpose(0, 2, 1)
  tab = tables.reshape(F * V, E)
  wv = W.reshape(F, E)
  bv = jnp.broadcast_to(b, (E,)).astype(jnp.float32)
  out = _run(xp, tab, wv, bv)
  return out.reshape(B, 1)
